# baseline (speedup 1.0000x reference)
import jax
import jax.numpy as jnp
from jax.experimental import pallas as pl
from jax.experimental.pallas import tpu as pltpu

_BATCH = 4
_ROWS = 8192
_DIM = 256
_CHUNKS = (128, 256, 512, 1024, 2048, 2048, 1024, 512, 256, 192, 128, 64)
_OFFS = tuple(sum(_CHUNKS[:i]) for i in range(len(_CHUNKS)))
assert sum(_CHUNKS) == _ROWS


def _fanout_body(w_hbm, out_hbm, buf, in_sems, out_sems):
    in_copies = [
        pltpu.make_async_copy(
            w_hbm.at[pl.ds(off, n), :],
            buf.at[pl.ds(off, n), :],
            in_sems.at[i],
        )
        for i, (off, n) in enumerate(zip(_OFFS, _CHUNKS))
    ]
    for c in in_copies:
        c.start()
    out_copies = []
    for i, (off, n) in enumerate(zip(_OFFS, _CHUNKS)):
        in_copies[i].wait()
        for b in range(_BATCH):
            c = pltpu.make_async_copy(
                buf.at[pl.ds(off, n), :],
                out_hbm.at[b, pl.ds(off, n), :],
                out_sems.at[i, b],
            )
            c.start()
            out_copies.append(c)
    for c in out_copies:
        c.wait()


def kernel(tokens, W):
    del tokens
    return pl.pallas_call(
        _fanout_body,
        in_specs=[pl.BlockSpec(memory_space=pl.ANY)],
        out_specs=pl.BlockSpec(memory_space=pl.ANY),
        out_shape=jax.ShapeDtypeStruct((_BATCH, _ROWS, _DIM), jnp.float32),
        scratch_shapes=[
            pltpu.VMEM((_ROWS, _DIM), jnp.float32),
            pltpu.SemaphoreType.DMA((len(_CHUNKS),)),
            pltpu.SemaphoreType.DMA((len(_CHUNKS), _BATCH)),
        ],
    )(W)
